# Initial kernel scaffold; baseline (speedup 1.0000x reference)
import jax
import jax.numpy as jnp
from jax.experimental import pallas as pl


def kernel(x, edge_index, edge_types, params):
    raise NotImplementedError("write your pallas kernel here")


# speedup vs baseline: 3.2113x; 3.2113x over previous
import functools
import math

import jax
import jax.numpy as jnp
from jax import lax
from jax.experimental import pallas as pl
from jax.experimental.pallas import tpu as pltpu
from jax.experimental.pallas import tpu_sc as plsc

N = 10000
E = 160000
D_IN = 256
D_H = 128
NL = 3
NT = 2

NSC = 2
NSUB = 16
HALF = N // NSC
HALF_PAD = 5120
CPR = HALF_PAD // NSUB
TRASH = NT * HALF_PAD
ROWS_PAD = 10496
ZROWS = ROWS_PAD // NSUB
EC = E // NSUB
CH = 128
NCHUNK = (EC + CH - 1) // CH
ECP = NCHUNK * CH

BLK = 1000
NB = N // BLK
BN_SCALE = 1.0 / math.sqrt(1.0 + 1e-5)

_mesh = plsc.VectorSubcoreMesh(core_axis_name="c", subcore_axis_name="s")


@functools.partial(
    pl.kernel,
    mesh=_mesh,
    out_type=jax.ShapeDtypeStruct((NSC, NT, HALF_PAD, D_H), jnp.float32),
    scratch_types=[
        pltpu.VMEM((NCHUNK, CH), jnp.int32),
        pltpu.VMEM((NCHUNK, CH), jnp.int32),
        pltpu.VMEM((CH, D_H), jnp.float32),
        pltpu.VMEM_SHARED((ROWS_PAD, D_H), jnp.float32),
        pltpu.SemaphoreType.DMA,
    ],
)
def _sc_aggregate(h_hbm, src_hbm, lidx_hbm, zeros_hbm, out_hbm,
                  src_v, lidx_v, rows_v, ash, sem):
    c = lax.axis_index("c")
    s = lax.axis_index("s")
    pltpu.sync_copy(zeros_hbm, ash.at[pl.ds(s * ZROWS, ZROWS)])
    pltpu.sync_copy(src_hbm.at[s], src_v)
    pltpu.sync_copy(lidx_hbm.at[c, s], lidx_v)
    plsc.subcore_barrier()

    def body(j, carry):
        pltpu.async_copy(h_hbm.at[src_v.at[j]], rows_v, sem).wait()
        pltpu.sync_copy(rows_v, ash.at[lidx_v.at[j]], add=True)
        return carry

    lax.fori_loop(0, NCHUNK, body, 0)
    plsc.subcore_barrier()
    for t in range(NT):
        pltpu.sync_copy(ash.at[pl.ds(t * HALF_PAD + s * CPR, CPR)],
                        out_hbm.at[c, t, pl.ds(s * CPR, CPR)])


@functools.partial(
    pl.kernel,
    mesh=_mesh,
    out_type=jax.ShapeDtypeStruct((NSC, NT, HALF_PAD, D_H), jnp.float32),
    scratch_types=[
        pltpu.VMEM((NCHUNK, CH), jnp.int32),
        pltpu.VMEM((CH, D_H), jnp.float32),
        pltpu.VMEM_SHARED((ROWS_PAD, D_H), jnp.float32),
        pltpu.SemaphoreType.DMA,
    ],
)
def _sc_counts(lidx_hbm, zeros_hbm, ones_hbm, out_hbm,
               lidx_v, ones_v, csh, sem):
    c = lax.axis_index("c")
    s = lax.axis_index("s")
    pltpu.sync_copy(zeros_hbm, csh.at[pl.ds(s * ZROWS, ZROWS)])
    pltpu.sync_copy(ones_hbm, ones_v)
    pltpu.sync_copy(lidx_hbm.at[c, s], lidx_v)
    plsc.subcore_barrier()

    def body(j, carry):
        pltpu.sync_copy(ones_v, csh.at[lidx_v.at[j]], add=True)
        return carry

    lax.fori_loop(0, NCHUNK, body, 0)
    plsc.subcore_barrier()
    for t in range(NT):
        pltpu.sync_copy(csh.at[pl.ds(t * HALF_PAD + s * CPR, CPR)],
                        out_hbm.at[c, t, pl.ds(s * CPR, CPR)])


def _dot(a, b):
    return lax.dot_general(a, b, (((1,), (1,)), ((), ())),
                           preferred_element_type=jnp.float32)


def _tc_proj_body(x_ref, wp_ref, bp_ref, out_ref):
    out_ref[...] = _dot(x_ref[...], wp_ref[...]) + bp_ref[...]


def _tc_proj(x, wp, bp):
    return pl.pallas_call(
        _tc_proj_body,
        grid=(NB,),
        in_specs=[
            pl.BlockSpec((BLK, D_IN), lambda i: (i, 0)),
            pl.BlockSpec((D_H, D_IN), lambda i: (0, 0)),
            pl.BlockSpec((1, D_H), lambda i: (0, 0)),
        ],
        out_specs=pl.BlockSpec((BLK, D_H), lambda i: (i, 0)),
        out_shape=jax.ShapeDtypeStruct((N, D_H), jnp.float32),
    )(x, wp, bp)


def _tc_layer_body(h_ref, a0_ref, a1_ref, c0_ref, c1_ref,
                   we0_ref, we1_ref, ws_ref,
                   be0_ref, be1_ref, bs_ref, gm_ref, bt_ref,
                   out_ref, *, bn):
    acc = (_dot(a0_ref[0, 0], we0_ref[...]) +
           _dot(a1_ref[0, 0], we1_ref[...]) +
           _dot(h_ref[...], ws_ref[...]))
    c0 = c0_ref[0, 0][:, 0:1]
    c1 = c1_ref[0, 0][:, 0:1]
    acc = acc + c0 * be0_ref[...] + c1 * be1_ref[...] + bs_ref[...]
    deg = c0 + c1
    deg = jnp.where(deg == 0.0, 1.0, deg)
    h = jnp.maximum(acc / deg, 0.0)
    if bn:
        h = h * (gm_ref[...] * BN_SCALE) + bt_ref[...]
        h = jnp.maximum(h, 0.0)
    out_ref[...] = h


_NBH = HALF // BLK


def _tc_layer(h, agg, cnt, lp, bnp):
    bn = bnp is not None
    gm = bnp['gamma'] if bn else lp['bs']
    bt = bnp['beta'] if bn else lp['bs']
    row = lambda v: v.reshape(1, D_H)
    full = lambda: pl.BlockSpec((D_H, D_H), lambda i: (0, 0))
    vec = lambda: pl.BlockSpec((1, D_H), lambda i: (0, 0))
    piece = lambda t: pl.BlockSpec(
        (1, 1, BLK, D_H), lambda i, t=t: (i // _NBH, t, i % _NBH, 0))
    return pl.pallas_call(
        functools.partial(_tc_layer_body, bn=bn),
        grid=(NB,),
        in_specs=[pl.BlockSpec((BLK, D_H), lambda i: (i, 0)),
                  piece(0), piece(1), piece(0), piece(1),
                  full(), full(), full(),
                  vec(), vec(), vec(), vec(), vec()],
        out_specs=pl.BlockSpec((BLK, D_H), lambda i: (i, 0)),
        out_shape=jax.ShapeDtypeStruct((N, D_H), jnp.float32),
    )(h, agg, agg, cnt, cnt,
      lp['We'][0], lp['We'][1], lp['Ws'],
      row(lp['be'][0]), row(lp['be'][1]), row(lp['bs']), row(gm), row(bt))


def _tc_head_body(h_ref, wr1_ref, br1_ref, wr2_ref, br2_ref, out_ref, acc_ref):
    i = pl.program_id(0)

    @pl.when(i == 0)
    def _():
        acc_ref[...] = jnp.zeros((8, D_H), jnp.float32)

    acc_ref[...] = acc_ref[...] + jnp.sum(h_ref[...], axis=0, keepdims=True)

    @pl.when(i == NB - 1)
    def _():
        g = acc_ref[0:1, :] * (1.0 / N)
        z = jnp.maximum(_dot(g, wr1_ref[...]) + br1_ref[...], 0.0)
        out_ref[...] = _dot(z, wr2_ref[...]) + br2_ref[...]


def _tc_head(h, wr1, br1, wr2, br2):
    return pl.pallas_call(
        _tc_head_body,
        grid=(NB,),
        in_specs=[
            pl.BlockSpec((BLK, D_H), lambda i: (i, 0)),
            pl.BlockSpec((D_H, D_H), lambda i: (0, 0)),
            pl.BlockSpec((1, D_H), lambda i: (0, 0)),
            pl.BlockSpec((D_H, D_H), lambda i: (0, 0)),
            pl.BlockSpec((1, D_H), lambda i: (0, 0)),
        ],
        out_specs=pl.BlockSpec((1, D_H), lambda i: (0, 0)),
        out_shape=jax.ShapeDtypeStruct((1, D_H), jnp.float32),
        scratch_shapes=[pltpu.VMEM((8, D_H), jnp.float32)],
    )(h, wr1, br1.reshape(1, D_H), wr2, br2.reshape(1, D_H))


def kernel(x, edge_index, edge_types, params):
    src = edge_index[0].astype(jnp.int32)
    dst = edge_index[1].astype(jnp.int32)
    et = edge_types.astype(jnp.int32)

    owner = dst // HALF
    lrow = et * HALF_PAD + (dst % HALF)
    lidx = jnp.stack([jnp.where(owner == c, lrow, TRASH) for c in range(NSC)])

    srcw = jnp.pad(src.reshape(NSUB, EC),
                   ((0, 0), (0, ECP - EC))).reshape(NSUB, NCHUNK, CH)
    lidxw = jnp.pad(lidx.reshape(NSC, NSUB, EC),
                    ((0, 0), (0, 0), (0, ECP - EC)),
                    constant_values=TRASH).reshape(NSC, NSUB, NCHUNK, CH)

    zeros_big = jnp.zeros((ZROWS, D_H), jnp.float32)
    ones_rows = jnp.ones((CH, D_H), jnp.float32)

    cnt = _sc_counts(lidxw, zeros_big, ones_rows)

    p = params
    h = _tc_proj(x, p['Wp'], p['bp'].reshape(1, D_H))
    for i in range(NL):
        agg = _sc_aggregate(h, srcw, lidxw, zeros_big)
        bnp = p['bn'][i] if i < NL - 1 else None
        h = _tc_layer(h, agg, cnt, p['layers'][i], bnp)
    return _tc_head(h, p['Wr1'], p['br1'], p['Wr2'], p['br2'])
